# baseline (speedup 1.0000x reference)
import functools
import jax
import jax.numpy as jnp
from jax import lax
from jax.experimental import pallas as pl
from jax.experimental.pallas import tpu as pltpu
from jax.experimental.pallas import tpu_sc as plsc

NUM_FIELDS = 26
VOCAB = 100000
EMBED_DIM = 16
BATCH = 16384

_TOT = BATCH * NUM_FIELDS
_NW = 32
_RPW = _TOT // _NW
_G = 128
_NG = _RPW // _G
_L = 16


def _sc_body(table_hbm, idx_hbm, out_hbm, idx_v, rows_a, rows_b, sem_a, sem_b):
    wid = lax.axis_index("s") * 2 + lax.axis_index("c")
    base = wid * _RPW

    pltpu.sync_copy(idx_hbm.at[pl.ds(wid * _NG, _NG)], idx_v)

    iota = lax.iota(jnp.int32, _L)

    @pl.loop(0, _NG)
    def _adjust(g):
        row0 = base + g * _G
        for v in range(_G // _L):
            pos = row0 + v * _L + iota
            off = lax.rem(pos, NUM_FIELDS) * VOCAB
            sl = (g, pl.ds(v * _L, _L))
            idx_v[sl] = idx_v[sl] + off

    bufs = (rows_a, rows_b)
    sems = (sem_a, sem_b)

    cp0 = pltpu.make_async_copy(table_hbm.at[idx_v.at[0]], rows_a, sem_a)
    cp0.start()

    @pl.loop(0, _NG)
    def _gather(g):
        for b in range(2):
            @pl.when(lax.rem(g, 2) == b)
            def _():
                nxt = (b + 1) % 2
                @pl.when(g + 1 < _NG)
                def _():
                    pltpu.make_async_copy(
                        table_hbm.at[idx_v.at[g + 1]], bufs[nxt], sems[nxt]
                    ).start()
                pltpu.make_async_copy(
                    table_hbm.at[idx_v.at[g]], bufs[b], sems[b]
                ).wait()
                pltpu.sync_copy(bufs[b], out_hbm.at[pl.ds(base + g * _G, _G)])


@functools.partial(jax.jit, static_argnames=())
def kernel(x_cat, tables):
    idx2d = x_cat.astype(jnp.int32).reshape(_TOT // _G, _G)
    table_flat = tables.reshape(NUM_FIELDS * VOCAB, EMBED_DIM)

    mesh = plsc.VectorSubcoreMesh(core_axis_name="c", subcore_axis_name="s")
    run = pl.kernel(
        _sc_body,
        out_type=jax.ShapeDtypeStruct((_TOT, EMBED_DIM), jnp.float32),
        mesh=mesh,
        scratch_types=[
            pltpu.VMEM((_NG, _G), jnp.int32),
            pltpu.VMEM((_G, EMBED_DIM), jnp.float32),
            pltpu.VMEM((_G, EMBED_DIM), jnp.float32),
            pltpu.SemaphoreType.DMA,
            pltpu.SemaphoreType.DMA,
        ],
        compiler_params=pltpu.CompilerParams(use_tc_tiling_on_sc=False),
    )
    out = run(table_flat, idx2d)
    return out.reshape(BATCH, NUM_FIELDS * EMBED_DIM)
